# baseline (speedup 1.0000x reference)
import functools

import jax
import jax.numpy as jnp
from jax import lax
from jax.experimental import pallas as pl
from jax.experimental.pallas import tpu as pltpu
from jax.experimental.pallas import tpu_sc as plsc

_B, _T, _D = 4, 4096, 2048
_W = 128
_K = 1024

_TSC = 1024
_TTC = _T - _TSC
_TT = 256
_NT = _TTC // _TT

_L = 16
_NW = 32
_WPB = _NW // _B
_TPW = _TSC // _WPB
_SLAB = 8
_NSLAB = _TPW // _SLAB
_DTILES = _D // 128


def _sc_stats_body(a_hbm, p_hbm, st_hbm, ch_hbm, abuf, pbuf, ost, och, sem):
    c = lax.axis_index("c")
    s = lax.axis_index("s")
    wid = s * 2 + c
    b = wid // _WPB
    tloc = (wid % _WPB) * _TPW

    zero = jnp.zeros((_L,), jnp.float32)
    row_ids = lax.broadcasted_iota(jnp.int32, (_L,), 0)

    def copies(slab, slot):
        t0 = _TTC + tloc + slab * _SLAB
        return (
            pltpu.make_async_copy(
                a_hbm.at[b, pl.ds(t0, _SLAB), :], abuf.at[slot], sem),
            pltpu.make_async_copy(
                p_hbm.at[b, pl.ds(t0, _SLAB), :], pbuf.at[slot], sem),
        )

    for h in copies(0, 0):
        h.start()

    def slab_body(slab, carry):
        slot = lax.rem(slab, 2)
        for h in copies(slab, slot):
            h.wait()

        @pl.when(slab < _NSLAB - 1)
        def _():
            for h in copies(slab + 1, 1 - slot):
                h.start()

        for r in range(_SLAB):
            def dbody(i, accs):
                asts, achs = accs
                nasts, nachs = [], []
                for l in range(8):
                    va = abuf[slot, r, pl.ds(i * 128 + l * _L, _L)]
                    vp = pbuf[slot, r, pl.ds(i * 128 + l * _L, _L)]
                    d = va - vp
                    nasts.append(asts[l] + va * va)
                    nachs.append(achs[l] + d * d)
                return tuple(nasts), tuple(nachs)
            asts, achs = lax.fori_loop(
                0, _DTILES, dbody, ((zero,) * 8, (zero,) * 8))

            def tree8(v):
                return ((v[0] + v[1]) + (v[2] + v[3])) + \
                       ((v[4] + v[5]) + (v[6] + v[7]))
            ast = tree8(asts)
            ach = tree8(achs)
            col = jnp.full((_L,), slab * _SLAB + r, jnp.int32)
            plsc.store_scatter(ost, [row_ids, col], ast)
            plsc.store_scatter(och, [row_ids, col], ach)
        return carry

    lax.fori_loop(0, _NSLAB, slab_body, 0)
    pltpu.sync_copy(ost, st_hbm.at[b, :, pl.ds(tloc, _TPW)])
    pltpu.sync_copy(och, ch_hbm.at[b, :, pl.ds(tloc, _TPW)])


_sc_stats = functools.partial(
    pl.kernel,
    _sc_stats_body,
    out_type=[
        jax.ShapeDtypeStruct((_B, _L, _TSC), jnp.float32),
        jax.ShapeDtypeStruct((_B, _L, _TSC), jnp.float32),
    ],
    mesh=plsc.VectorSubcoreMesh(core_axis_name="c", subcore_axis_name="s"),
    compiler_params=pltpu.CompilerParams(
        use_tc_tiling_on_sc=True, needs_layout_passes=False),
    scratch_types=[
        pltpu.VMEM((2, _SLAB, _D), jnp.float32),
        pltpu.VMEM((2, _SLAB, _D), jnp.float32),
        pltpu.VMEM((_L, _TPW), jnp.float32),
        pltpu.VMEM((_L, _TPW), jnp.float32),
        pltpu.SemaphoreType.DMA,
    ],
)


def _tc_stats_body(a_ref, p_ref, dst_ref, dch_ref):
    a = a_ref[...]
    p = p_ref[...]
    inv_d = jnp.float32(1.0 / _D)
    d = a - p
    dst_ref[...] = jnp.sum(a * a, axis=-1) * inv_d
    dch_ref[...] = jnp.sum(d * d, axis=-1) * inv_d


def _shift_right(x, s):
    return jnp.concatenate(
        [jnp.zeros((x.shape[0], s), x.dtype), x[:, : x.shape[1] - s]], axis=1)


def _prefix_sum(x):
    n = x.shape[1]
    s = 1
    while s < n:
        x = x + _shift_right(x, s)
        s *= 2
    return x


def _window_sum(x):
    s = 1
    while s < _W:
        x = x + _shift_right(x, s)
        s *= 2
    return x


def _routing_body(scal_ref, dst_tc_ref, dch_tc_ref, sc_st_ref, sc_ch_ref,
                  g_ref, m_ref):
    c_ce = scal_ref[0]
    m_cu = scal_ref[1]
    bce = scal_ref[2]
    bcu = scal_ref[3]
    inv_d = jnp.float32(1.0 / _D)

    sc_st = jnp.sum(sc_st_ref[...], axis=1) * inv_d
    sc_ch = jnp.sum(sc_ch_ref[...], axis=1) * inv_d
    d_st = jnp.concatenate([dst_tc_ref[...], sc_st], axis=1)
    d_ch = jnp.concatenate([dch_tc_ref[...], sc_ch], axis=1)

    ce = d_st - (d_ch - c_ce)
    wsum = _window_sum(d_st)
    pos = jax.lax.broadcasted_iota(jnp.int32, (_B, _T), 1).astype(jnp.float32)
    counts = jnp.minimum(pos + 1.0, jnp.float32(_W))
    cu = d_st - m_cu * (wsum / counts)

    s_ce = 1.0 / (1.0 + jnp.exp(-bce * ce))
    s_cu = 1.0 / (1.0 + jnp.exp(-bcu * cu))
    g = s_ce + s_cu - s_ce * s_cu
    g_ref[...] = g

    bits = jax.lax.bitcast_convert_type(g, jnp.int32)
    lo = jnp.min(bits, axis=1, keepdims=True) - 1
    hi = jnp.max(bits, axis=1, keepdims=True) + 1

    def body(carry):
        lo, hi = carry
        mid = lo + (hi - lo) // 2
        cnt = jnp.sum((bits >= mid).astype(jnp.int32), axis=1, keepdims=True)
        ge = cnt >= _K
        return jnp.where(ge, mid, lo), jnp.where(ge, hi, mid)

    lo, hi = jax.lax.while_loop(
        lambda c: jnp.any(c[1] - c[0] > 1), body, (lo, hi))
    tau = lo
    gt = bits > tau
    eq = bits == tau
    cnt_gt = jnp.sum(gt.astype(jnp.int32), axis=1, keepdims=True)
    need = _K - cnt_gt
    eq_rank = _prefix_sum(eq.astype(jnp.int32))
    mask = gt | (eq & (eq_rank <= need))
    m_ref[...] = mask.astype(jnp.float32)


def kernel(actual_residual, predicted_residual, raw_o_ce, raw_m_cu, beta_ce, beta_cu):
    sc_st, sc_ch = _sc_stats()(actual_residual, predicted_residual)

    dst_tc, dch_tc = pl.pallas_call(
        _tc_stats_body,
        grid=(_NT,),
        in_specs=[
            pl.BlockSpec((_B, _TT, _D), lambda t: (0, t, 0)),
            pl.BlockSpec((_B, _TT, _D), lambda t: (0, t, 0)),
        ],
        out_specs=[
            pl.BlockSpec((_B, _TT), lambda t: (0, t)),
            pl.BlockSpec((_B, _TT), lambda t: (0, t)),
        ],
        out_shape=[
            jax.ShapeDtypeStruct((_B, _TTC), jnp.float32),
            jax.ShapeDtypeStruct((_B, _TTC), jnp.float32),
        ],
    )(actual_residual, predicted_residual)

    o_ce_pos = jax.nn.softplus(jnp.asarray(raw_o_ce, jnp.float32))
    m_cu_pos = jax.nn.softplus(jnp.asarray(raw_m_cu, jnp.float32))
    scal = jnp.stack([
        jnp.log(o_ce_pos + 1e-10),
        m_cu_pos,
        jnp.asarray(beta_ce, jnp.float32),
        jnp.asarray(beta_cu, jnp.float32),
    ])

    g, mask = pl.pallas_call(
        _routing_body,
        in_specs=[
            pl.BlockSpec(memory_space=pltpu.SMEM),
            pl.BlockSpec(memory_space=pltpu.VMEM),
            pl.BlockSpec(memory_space=pltpu.VMEM),
            pl.BlockSpec(memory_space=pltpu.VMEM),
            pl.BlockSpec(memory_space=pltpu.VMEM),
        ],
        out_specs=[
            pl.BlockSpec(memory_space=pltpu.VMEM),
            pl.BlockSpec(memory_space=pltpu.VMEM),
        ],
        out_shape=[
            jax.ShapeDtypeStruct((_B, _T), jnp.float32),
            jax.ShapeDtypeStruct((_B, _T), jnp.float32),
        ],
    )(scal, dst_tc, dch_tc, sc_st, sc_ch)
    return (g, mask)


# speedup vs baseline: 1.1777x; 1.1777x over previous
import jax
import jax.numpy as jnp
from jax.experimental import pallas as pl
from jax.experimental.pallas import tpu as pltpu

_B, _T, _D = 4, 4096, 2048
_W = 128
_K = 1024
_TT = 256
_NT = _T // _TT


def _shift_right(x, s):
    return jnp.concatenate(
        [jnp.zeros((x.shape[0], s), x.dtype), x[:, : x.shape[1] - s]], axis=1)


def _window_sum(x):
    s = 1
    while s < _W:
        x = x + _shift_right(x, s)
        s *= 2
    return x


def _prefix_sum(x):
    n = x.shape[1]
    s = 1
    while s < n:
        x = x + _shift_right(x, s)
        s *= 2
    return x


def _routing(scal_ref, d_st, d_ch, g_ref, m_ref):
    c_ce = scal_ref[0]
    m_cu = scal_ref[1]
    bce = scal_ref[2]
    bcu = scal_ref[3]

    ce = d_st - (d_ch - c_ce)
    wsum = _window_sum(d_st)
    pos = jax.lax.broadcasted_iota(jnp.int32, (_B, _T), 1).astype(jnp.float32)
    counts = jnp.minimum(pos + 1.0, jnp.float32(_W))
    cu = d_st - m_cu * (wsum / counts)

    s_ce = 1.0 / (1.0 + jnp.exp(-bce * ce))
    s_cu = 1.0 / (1.0 + jnp.exp(-bcu * cu))
    g = s_ce + s_cu - s_ce * s_cu
    g_ref[...] = g

    bits = jax.lax.bitcast_convert_type(g, jnp.int32)
    lo = jnp.min(bits, axis=1, keepdims=True) - 1
    hi = jnp.max(bits, axis=1, keepdims=True) + 1

    def count_ge(t):
        return jnp.sum((bits >= t).astype(jnp.int32), axis=1, keepdims=True)

    def body(carry):
        lo, hi = carry
        d = jnp.maximum((hi - lo + 3) // 4, 1)
        t1 = jnp.minimum(lo + d, hi - 1)
        t2 = jnp.minimum(lo + 2 * d, hi - 1)
        t3 = jnp.minimum(lo + 3 * d, hi - 1)
        g1 = count_ge(t1) >= _K
        g2 = count_ge(t2) >= _K
        g3 = count_ge(t3) >= _K
        nlo = jnp.where(g3, t3, jnp.where(g2, t2, jnp.where(g1, t1, lo)))
        nhi = jnp.where(~g1, t1, jnp.where(~g2, t2, jnp.where(~g3, t3, hi)))
        return nlo, nhi

    lo, hi = jax.lax.while_loop(
        lambda c: jnp.any(c[1] - c[0] > 1), body, (lo, hi))
    tau = lo
    gt = bits > tau
    eq = bits == tau
    cnt_gt = jnp.sum(gt.astype(jnp.int32), axis=1, keepdims=True)
    need = _K - cnt_gt
    eq_rank = _prefix_sum(eq.astype(jnp.int32))
    mask = gt | (eq & (eq_rank <= need))
    m_ref[...] = mask.astype(jnp.float32)


def _fused_body(scal_ref, a_ref, p_ref, g_ref, m_ref, dst_s, dch_s):
    t = pl.program_id(0)
    a = a_ref[...]
    p = p_ref[...]
    inv_d = jnp.float32(1.0 / _D)
    d = a - p
    dst_s[:, pl.ds(t * _TT, _TT)] = jnp.sum(a * a, axis=-1) * inv_d
    dch_s[:, pl.ds(t * _TT, _TT)] = jnp.sum(d * d, axis=-1) * inv_d

    @pl.when(t == _NT - 1)
    def _():
        _routing(scal_ref, dst_s[...], dch_s[...], g_ref, m_ref)


def kernel(actual_residual, predicted_residual, raw_o_ce, raw_m_cu, beta_ce, beta_cu):
    o_ce_pos = jax.nn.softplus(jnp.asarray(raw_o_ce, jnp.float32))
    m_cu_pos = jax.nn.softplus(jnp.asarray(raw_m_cu, jnp.float32))
    scal = jnp.stack([
        jnp.log(o_ce_pos + 1e-10),
        m_cu_pos,
        jnp.asarray(beta_ce, jnp.float32),
        jnp.asarray(beta_cu, jnp.float32),
    ])

    g, mask = pl.pallas_call(
        _fused_body,
        grid=(_NT,),
        in_specs=[
            pl.BlockSpec(memory_space=pltpu.SMEM),
            pl.BlockSpec((_B, _TT, _D), lambda t: (0, t, 0)),
            pl.BlockSpec((_B, _TT, _D), lambda t: (0, t, 0)),
        ],
        out_specs=[
            pl.BlockSpec((_B, _T), lambda t: (0, 0)),
            pl.BlockSpec((_B, _T), lambda t: (0, 0)),
        ],
        out_shape=[
            jax.ShapeDtypeStruct((_B, _T), jnp.float32),
            jax.ShapeDtypeStruct((_B, _T), jnp.float32),
        ],
        scratch_shapes=[
            pltpu.VMEM((_B, _T), jnp.float32),
            pltpu.VMEM((_B, _T), jnp.float32),
        ],
    )(scal, actual_residual, predicted_residual)
    return (g, mask)


# speedup vs baseline: 1.1937x; 1.0136x over previous
import jax
import jax.numpy as jnp
from jax.experimental import pallas as pl
from jax.experimental.pallas import tpu as pltpu

_B, _T, _D = 4, 4096, 2048
_W = 128
_K = 1024
_TT = 256
_NT = _T // _TT


def _shift_right(x, s):
    return jnp.concatenate(
        [jnp.zeros((x.shape[0], s), x.dtype), x[:, : x.shape[1] - s]], axis=1)


def _window_sum(x):
    s = 1
    while s < _W:
        x = x + _shift_right(x, s)
        s *= 2
    return x


def _prefix_sum(x):
    n = x.shape[1]
    s = 1
    while s < n:
        x = x + _shift_right(x, s)
        s *= 2
    return x


def _routing(scal_ref, d_st, d_ch, g_ref, m_ref):
    c_ce = scal_ref[0]
    m_cu = scal_ref[1]
    bce = scal_ref[2]
    bcu = scal_ref[3]

    ce = d_st - (d_ch - c_ce)
    wsum = _window_sum(d_st)
    pos = jax.lax.broadcasted_iota(jnp.int32, (_B, _T), 1).astype(jnp.float32)
    counts = jnp.minimum(pos + 1.0, jnp.float32(_W))
    cu = d_st - m_cu * (wsum / counts)

    s_ce = 1.0 / (1.0 + jnp.exp(-bce * ce))
    s_cu = 1.0 / (1.0 + jnp.exp(-bcu * cu))
    g = s_ce + s_cu - s_ce * s_cu
    g_ref[...] = g

    bits = jax.lax.bitcast_convert_type(g, jnp.int32)
    lo = jnp.min(bits, axis=1, keepdims=True) - 1
    hi = jnp.max(bits, axis=1, keepdims=True) + 1

    def body(carry):
        lo, hi = carry
        mid = lo + (hi - lo) // 2
        cnt = jnp.sum((bits >= mid).astype(jnp.int32), axis=1, keepdims=True)
        ge = cnt >= _K
        return jnp.where(ge, mid, lo), jnp.where(ge, hi, mid)

    lo, hi = jax.lax.while_loop(
        lambda c: jnp.any(c[1] - c[0] > 1), body, (lo, hi))
    tau = lo
    gt = bits > tau
    eq = bits == tau
    cnt_gt = jnp.sum(gt.astype(jnp.int32), axis=1, keepdims=True)
    need = _K - cnt_gt
    eq_rank = _prefix_sum(eq.astype(jnp.int32))
    mask = gt | (eq & (eq_rank <= need))
    m_ref[...] = mask.astype(jnp.float32)


def _fused_body(scal_ref, a_ref, p_ref, g_ref, m_ref, dst_s, dch_s):
    t = pl.program_id(0)
    a = a_ref[...]
    p = p_ref[...]
    inv_d = jnp.float32(1.0 / _D)
    d = a - p
    dst_s[:, pl.ds(t * _TT, _TT)] = jnp.sum(a * a, axis=-1) * inv_d
    dch_s[:, pl.ds(t * _TT, _TT)] = jnp.sum(d * d, axis=-1) * inv_d

    @pl.when(t == _NT - 1)
    def _():
        _routing(scal_ref, dst_s[...], dch_s[...], g_ref, m_ref)


def kernel(actual_residual, predicted_residual, raw_o_ce, raw_m_cu, beta_ce, beta_cu):
    o_ce_pos = jax.nn.softplus(jnp.asarray(raw_o_ce, jnp.float32))
    m_cu_pos = jax.nn.softplus(jnp.asarray(raw_m_cu, jnp.float32))
    scal = jnp.stack([
        jnp.log(o_ce_pos + 1e-10),
        m_cu_pos,
        jnp.asarray(beta_ce, jnp.float32),
        jnp.asarray(beta_cu, jnp.float32),
    ])

    g, mask = pl.pallas_call(
        _fused_body,
        grid=(_NT,),
        in_specs=[
            pl.BlockSpec(memory_space=pltpu.SMEM),
            pl.BlockSpec((_B, _TT, _D), lambda t: (0, t, 0)),
            pl.BlockSpec((_B, _TT, _D), lambda t: (0, t, 0)),
        ],
        out_specs=[
            pl.BlockSpec((_B, _T), lambda t: (0, 0)),
            pl.BlockSpec((_B, _T), lambda t: (0, 0)),
        ],
        out_shape=[
            jax.ShapeDtypeStruct((_B, _T), jnp.float32),
            jax.ShapeDtypeStruct((_B, _T), jnp.float32),
        ],
        scratch_shapes=[
            pltpu.VMEM((_B, _T), jnp.float32),
            pltpu.VMEM((_B, _T), jnp.float32),
        ],
    )(scal, actual_residual, predicted_residual)
    return (g, mask)
